# Initial kernel scaffold; baseline (speedup 1.0000x reference)
import jax
import jax.numpy as jnp
from jax.experimental import pallas as pl


def kernel(feat, W1, W2, W_cls, neigh_idx, nodes):
    raise NotImplementedError("write your pallas kernel here")


# speedup vs baseline: 2.5166x; 2.5166x over previous
import functools

import jax
import jax.numpy as jnp
from jax import lax
from jax.experimental import pallas as pl
from jax.experimental.pallas import tpu as pltpu
from jax.experimental.pallas import tpu_sc as plsc

NC = 2
NSC = 16
NW = NC * NSC
L = 16


def _tc_project(feat, W1):
    n, d = feat.shape
    e = W1.shape[0]
    blk = 1000
    assert n % blk == 0

    def body(x_ref, w_ref, o_ref):
        o_ref[...] = lax.dot_general(
            x_ref[...], w_ref[...], (((1,), (1,)), ((), ())),
            preferred_element_type=jnp.float32)

    return pl.pallas_call(
        body,
        grid=(n // blk,),
        in_specs=[
            pl.BlockSpec((blk, d), lambda i: (i, 0)),
            pl.BlockSpec((e, d), lambda i: (0, 0)),
        ],
        out_specs=pl.BlockSpec((blk, e), lambda i: (i, 0)),
        out_shape=jax.ShapeDtypeStruct((n, e), jnp.float32),
    )(feat, W1)


def _tc_pad_neigh(neigh_idx, width):
    n, s = neigh_idx.shape
    blk = 1000
    assert n % blk == 0

    def body(x_ref, o_ref):
        o_ref[...] = jnp.concatenate(
            [x_ref[...], jnp.zeros((blk, width - s), jnp.int32)], axis=1)

    return pl.pallas_call(
        body,
        grid=(n // blk,),
        in_specs=[pl.BlockSpec((blk, s), lambda i: (i, 0))],
        out_specs=pl.BlockSpec((blk, width), lambda i: (i, 0)),
        out_shape=jax.ShapeDtypeStruct((n, width), jnp.int32),
    )(neigh_idx)


def _tc_head(P, W2, W_cls, scale):
    b2, e = P.shape
    c = W_cls.shape[0]

    def body(p_ref, w2_ref, wc_ref, o_ref):
        wc2 = lax.dot_general(
            wc_ref[...], w2_ref[...], (((1,), (0,)), ((), ())),
            preferred_element_type=jnp.float32)
        o_ref[...] = scale * lax.dot_general(
            p_ref[...], wc2, (((1,), (1,)), ((), ())),
            preferred_element_type=jnp.float32)

    return pl.pallas_call(
        body, out_shape=jax.ShapeDtypeStruct((b2, c), jnp.float32),
    )(P, W2, W_cls)


def _sc_aggregate(G, neigh_pad, nodes_flat, s):
    n, emb = G.shape
    nslot = nodes_flat.shape[0]
    nb = nslot // 2
    slots_w = nslot // NW
    bw = nb // NW
    ss = s * s
    ssp = 104
    nv = emb // L
    lvl1 = slots_w * s

    mesh = plsc.VectorSubcoreMesh(
        core_axis_name="c", subcore_axis_name="s",
        num_cores=NC, num_subcores=NSC)

    @functools.partial(
        pl.kernel,
        out_type=jax.ShapeDtypeStruct((nb, emb), jnp.float32),
        mesh=mesh,
        compiler_params=pltpu.CompilerParams(needs_layout_passes=False),
        scratch_types=[
            pltpu.VMEM((slots_w,), jnp.int32),
            pltpu.VMEM((slots_w, emb), jnp.int32),
            pltpu.VMEM((lvl1,), jnp.int32),
            pltpu.VMEM((slots_w, emb), jnp.int32),
            pltpu.VMEM((slots_w * ssp,), jnp.int32),
            pltpu.VMEM((ssp, emb), jnp.float32),
            pltpu.VMEM((bw, emb), jnp.float32),
            pltpu.SemaphoreType.DMA,
            pltpu.SemaphoreType.DMA,
        ],
    )
    def sc_kernel(g_hbm, ni_hbm, nodes_hbm, out_hbm,
                  nodes_v, nb2d, nb2f, nbd, nbf, grow, out_v, sem1, sem2):
        wid = lax.axis_index("s") * NC + lax.axis_index("c")
        base_slot = wid * slots_w

        pltpu.sync_copy(nodes_hbm.at[pl.ds(base_slot, slots_w)], nodes_v)
        pltpu.async_copy(ni_hbm.at[nodes_v], nb2d, sem1).wait()

        iota = lax.iota(jnp.int32, L)

        def div_s(x):
            assert s == 10
            return (x * 6554) >> 16

        def flat1(t, carry):
            k = t * L + iota
            row = div_s(k)
            col = k - row * s
            v = plsc.load_gather(nb2d, [row, col])
            nb2f[pl.ds(pl.multiple_of(t * L, L), L)] = v
            return carry
        lax.fori_loop(0, lvl1 // L, flat1, 0)

        zero16 = jnp.zeros((L,), jnp.int32)
        def fillpad(t, carry):
            r = t * L + iota
            for dc in range(ssp - ss):
                plsc.store_scatter(nbf, [r * ssp + (ss + dc)], zero16)
            return carry
        lax.fori_loop(0, slots_w // L, fillpad, 0)

        def lvl2(c, carry):
            idx = nb2f.at[pl.ds(pl.multiple_of(c * slots_w, 8), slots_w)]
            pltpu.async_copy(ni_hbm.at[idx], nbd, sem1).wait()

            def scat(t, carry2):
                k = t * L + iota
                j = div_s(k)
                sp = k - j * s
                m = c * slots_w + j
                i = div_s(m)
                s1 = m - i * s
                v = plsc.load_gather(nbd, [j, sp])
                plsc.store_scatter(nbf, [i * ssp + s1 * s + sp], v)
                return carry2
            lax.fori_loop(0, lvl1 // L, scat, 0)
            return carry
        lax.fori_loop(0, lvl1 // slots_w, lvl2, 0)

        def per_b(b, carry):
            acc = [jnp.zeros((L,), jnp.float32) for _ in range(nv)]
            for e in range(2):
                slot = b * 2 + e
                idx = nbf.at[pl.ds(pl.multiple_of(slot * ssp, 8), ssp)]
                pltpu.async_copy(g_hbm.at[idx], grow, sem2).wait()

                def per_s(si, acc_c):
                    part = [jnp.zeros((L,), jnp.float32) for _ in range(nv)]
                    for t in range(s):
                        r = si * s + t
                        for v in range(nv):
                            part[v] = part[v] + grow[r, pl.ds(v * L, L)]
                    return [a + jnp.maximum(p, 0.0)
                            for a, p in zip(acc_c, part)]
                acc = lax.fori_loop(0, s, per_s, acc)
            for v in range(nv):
                out_v[b, pl.ds(v * L, L)] = acc[v]
            return carry
        lax.fori_loop(0, bw, per_b, 0)

        pltpu.sync_copy(out_v, out_hbm.at[pl.ds(wid * bw, bw)])

    return sc_kernel(G, neigh_pad, nodes_flat)


def kernel(feat, W1, W2, W_cls, neigh_idx, nodes):
    s = neigh_idx.shape[1]
    G = _tc_project(feat, W1)
    ni_pad = _tc_pad_neigh(neigh_idx.astype(jnp.int32), G.shape[1])
    P = _sc_aggregate(G, ni_pad, nodes.reshape(-1).astype(jnp.int32), s)
    return _tc_head(P, W2, W_cls, 1.0 / (s * s * 2))
